# baseline (speedup 1.0000x reference)
import jax
import jax.numpy as jnp
from jax.experimental import pallas as pl
from jax.experimental.pallas import tpu as pltpu

N = 10000
D = 128
FT = 128
B = 64
ALPHA = 0.1
BLK = 200
NBLK = N // BLK


def _tdot(x, w):
    return jax.lax.dot_general(x, w, (((1,), (1,)), ((), ())),
                               preferred_element_type=jnp.float32)


def _body(cid_ref, pid_ref, adj_ref, cd_ref, init_ref, w1_ref, w2_ref,
          adj_hbm, init_hbm, pat_hbm, wih_ref, whh_ref, bih_ref, bhh_ref,
          feat_ref, td_ref, out_ref,
          xw1_s, ybuf, adj_sel, init_sel, pat_row, sem_a, sem_i, sem_p):
    i = pl.program_id(0)

    @pl.when(i == 0)
    def _prologue():
        for b in range(B):
            pltpu.make_async_copy(adj_hbm.at[pl.ds(cid_ref[b], 1), :],
                                  adj_sel.at[pl.ds(b, 1), :], sem_a).start()
            pltpu.make_async_copy(init_hbm.at[pl.ds(cid_ref[b], 1), :],
                                  init_sel.at[pl.ds(b, 1), :], sem_i).start()
        pltpu.make_async_copy(pat_hbm.at[pl.ds(pid_ref[0], 1), :],
                              pat_row.at[pl.ds(0, 1), :], sem_p).start()
        xw1_s[...] = jnp.dot(cd_ref[...], w1_ref[...],
                             preferred_element_type=jnp.float32)
        for b in range(B):
            pltpu.make_async_copy(init_hbm.at[pl.ds(0, 1), :],
                                  init_sel.at[pl.ds(b, 1), :], sem_i).wait()
            pltpu.make_async_copy(adj_hbm.at[pl.ds(0, 1), :],
                                  adj_sel.at[pl.ds(b, 1), :], sem_a).wait()
        pltpu.make_async_copy(pat_hbm.at[pl.ds(0, 1), :],
                              pat_row.at[pl.ds(0, 1), :], sem_p).wait()

    x1 = jnp.maximum(
        ALPHA * jnp.dot(adj_ref[...], xw1_s[...],
                        preferred_element_type=jnp.float32)
        + (1.0 - ALPHA) * init_ref[...], 0.0)
    ybuf[pl.ds(i * BLK, BLK), :] = jnp.dot(
        x1, w2_ref[...], preferred_element_type=jnp.float32)

    @pl.when(i == NBLK - 1)
    def _epilogue():
        s = jnp.dot(adj_sel[...], ybuf[...],
                    preferred_element_type=jnp.float32)
        x3 = jnp.maximum(ALPHA * s + (1.0 - ALPHA) * init_sel[...], 0.0)
        pe = jnp.broadcast_to(pat_row[0], (B, D))
        input1 = jnp.concatenate([pe, td_ref[...], feat_ref[...]], axis=1)
        pre = (_tdot(input1, wih_ref[...]) + bih_ref[...]
               + _tdot(x3, whh_ref[...]) + bhh_ref[...])
        h = jnp.tanh(pre)
        norm = jnp.sqrt(jnp.sum(h * h, axis=1, keepdims=True))
        out_ref[...] = h / jnp.maximum(norm, 1e-12)


def kernel(patient_dynamic, code_dynamic, init_code_dynamic, adj, features,
           timediffs, W1, W2, W_ih, W_hh, b_ih, b_hh,
           patientid, codeid, ancestorid):
    hbm = pl.BlockSpec(memory_space=pltpu.MemorySpace.HBM)
    out = pl.pallas_call(
        _body,
        grid_spec=pltpu.PrefetchScalarGridSpec(
            num_scalar_prefetch=2,
            grid=(NBLK,),
            in_specs=[
                pl.BlockSpec((BLK, N), lambda i, cid, pid: (i, 0)),
                pl.BlockSpec((N, D), lambda i, cid, pid: (0, 0)),
                pl.BlockSpec((BLK, D), lambda i, cid, pid: (i, 0)),
                pl.BlockSpec((D, D), lambda i, cid, pid: (0, 0)),
                pl.BlockSpec((D, D), lambda i, cid, pid: (0, 0)),
                hbm,
                hbm,
                hbm,
                pl.BlockSpec((D, D + 1 + FT),
                             lambda i, cid, pid: (0, 0)),
                pl.BlockSpec((D, D), lambda i, cid, pid: (0, 0)),
                pl.BlockSpec((1, D), lambda i, cid, pid: (0, 0)),
                pl.BlockSpec((1, D), lambda i, cid, pid: (0, 0)),
                pl.BlockSpec((B, FT), lambda i, cid, pid: (0, 0)),
                pl.BlockSpec((B, 1), lambda i, cid, pid: (0, 0)),
            ],
            out_specs=pl.BlockSpec((B, D), lambda i, cid, pid: (0, 0)),
            scratch_shapes=[
                pltpu.VMEM((N, D), jnp.float32),
                pltpu.VMEM((N, D), jnp.float32),
                pltpu.VMEM((B, N), jnp.float32),
                pltpu.VMEM((B, D), jnp.float32),
                pltpu.VMEM((1, D), jnp.float32),
                pltpu.SemaphoreType.DMA,
                pltpu.SemaphoreType.DMA,
                pltpu.SemaphoreType.DMA,
            ],
        ),
        out_shape=jax.ShapeDtypeStruct((B, D), jnp.float32),
    )(codeid.astype(jnp.int32), patientid.astype(jnp.int32),
      adj, code_dynamic, init_code_dynamic, W1, W2,
      adj, init_code_dynamic, patient_dynamic, W_ih, W_hh,
      b_ih.reshape(1, D), b_hh.reshape(1, D), features, timediffs)
    return out


# speedup vs baseline: 1.0111x; 1.0111x over previous
import jax
import jax.numpy as jnp
from jax.experimental import pallas as pl
from jax.experimental.pallas import tpu as pltpu

N = 10000
D = 128
FT = 128
B = 64
ALPHA = 0.1
BLK = 400
NBLK = N // BLK


def _tdot(x, w):
    return jax.lax.dot_general(x, w, (((1,), (1,)), ((), ())),
                               preferred_element_type=jnp.float32)


def _body(cid_ref, pid_ref, adj_ref, cd_ref, init_ref, w1_ref, w2_ref,
          adj_hbm, init_hbm, pat_hbm, wih_ref, whh_ref, bih_ref, bhh_ref,
          feat_ref, td_ref, out_ref,
          xw1_s, ybuf, adj_sel, init_sel, pat_row, sem_a, sem_i, sem_p):
    i = pl.program_id(0)

    @pl.when(i == 0)
    def _prologue():
        for b in range(B):
            pltpu.make_async_copy(adj_hbm.at[pl.ds(cid_ref[b], 1), :],
                                  adj_sel.at[pl.ds(b, 1), :], sem_a).start()
            pltpu.make_async_copy(init_hbm.at[pl.ds(cid_ref[b], 1), :],
                                  init_sel.at[pl.ds(b, 1), :], sem_i).start()
        pltpu.make_async_copy(pat_hbm.at[pl.ds(pid_ref[0], 1), :],
                              pat_row.at[pl.ds(0, 1), :], sem_p).start()
        xw1_s[...] = jnp.dot(cd_ref[...], w1_ref[...],
                             preferred_element_type=jnp.float32)

    x1 = jnp.maximum(
        ALPHA * jnp.dot(adj_ref[...], xw1_s[...],
                        preferred_element_type=jnp.float32)
        + (1.0 - ALPHA) * init_ref[...], 0.0)
    ybuf[pl.ds(i * BLK, BLK), :] = jnp.dot(
        x1, w2_ref[...], preferred_element_type=jnp.float32)

    @pl.when(i == NBLK - 1)
    def _epilogue():
        for b in range(B):
            pltpu.make_async_copy(init_hbm.at[pl.ds(0, 1), :],
                                  init_sel.at[pl.ds(b, 1), :], sem_i).wait()
            pltpu.make_async_copy(adj_hbm.at[pl.ds(0, 1), :],
                                  adj_sel.at[pl.ds(b, 1), :], sem_a).wait()
        pltpu.make_async_copy(pat_hbm.at[pl.ds(0, 1), :],
                              pat_row.at[pl.ds(0, 1), :], sem_p).wait()
        s = jnp.dot(adj_sel[...], ybuf[...],
                    preferred_element_type=jnp.float32)
        x3 = jnp.maximum(ALPHA * s + (1.0 - ALPHA) * init_sel[...], 0.0)
        pe = jnp.broadcast_to(pat_row[0], (B, D))
        input1 = jnp.concatenate([pe, td_ref[...], feat_ref[...]], axis=1)
        pre = (_tdot(input1, wih_ref[...]) + bih_ref[...]
               + _tdot(x3, whh_ref[...]) + bhh_ref[...])
        h = jnp.tanh(pre)
        norm = jnp.sqrt(jnp.sum(h * h, axis=1, keepdims=True))
        out_ref[...] = h / jnp.maximum(norm, 1e-12)


def kernel(patient_dynamic, code_dynamic, init_code_dynamic, adj, features,
           timediffs, W1, W2, W_ih, W_hh, b_ih, b_hh,
           patientid, codeid, ancestorid):
    hbm = pl.BlockSpec(memory_space=pltpu.MemorySpace.HBM)
    out = pl.pallas_call(
        _body,
        grid_spec=pltpu.PrefetchScalarGridSpec(
            num_scalar_prefetch=2,
            grid=(NBLK,),
            in_specs=[
                pl.BlockSpec((BLK, N), lambda i, cid, pid: (i, 0)),
                pl.BlockSpec((N, D), lambda i, cid, pid: (0, 0)),
                pl.BlockSpec((BLK, D), lambda i, cid, pid: (i, 0)),
                pl.BlockSpec((D, D), lambda i, cid, pid: (0, 0)),
                pl.BlockSpec((D, D), lambda i, cid, pid: (0, 0)),
                hbm,
                hbm,
                hbm,
                pl.BlockSpec((D, D + 1 + FT),
                             lambda i, cid, pid: (0, 0)),
                pl.BlockSpec((D, D), lambda i, cid, pid: (0, 0)),
                pl.BlockSpec((1, D), lambda i, cid, pid: (0, 0)),
                pl.BlockSpec((1, D), lambda i, cid, pid: (0, 0)),
                pl.BlockSpec((B, FT), lambda i, cid, pid: (0, 0)),
                pl.BlockSpec((B, 1), lambda i, cid, pid: (0, 0)),
            ],
            out_specs=pl.BlockSpec((B, D), lambda i, cid, pid: (0, 0)),
            scratch_shapes=[
                pltpu.VMEM((N, D), jnp.float32),
                pltpu.VMEM((N, D), jnp.float32),
                pltpu.VMEM((B, N), jnp.float32),
                pltpu.VMEM((B, D), jnp.float32),
                pltpu.VMEM((1, D), jnp.float32),
                pltpu.SemaphoreType.DMA,
                pltpu.SemaphoreType.DMA,
                pltpu.SemaphoreType.DMA,
            ],
        ),
        out_shape=jax.ShapeDtypeStruct((B, D), jnp.float32),
    )(codeid.astype(jnp.int32), patientid.astype(jnp.int32),
      adj, code_dynamic, init_code_dynamic, W1, W2,
      adj, init_code_dynamic, patient_dynamic, W_ih, W_hh,
      b_ih.reshape(1, D), b_hh.reshape(1, D), features, timediffs)
    return out
